# baseline (speedup 1.0000x reference)
import functools

import jax
import jax.numpy as jnp
from jax import lax
from jax.experimental import pallas as pl
from jax.experimental.pallas import tpu as pltpu
from jax.experimental.pallas import tpu_sc as plsc

BATCH = 16384
N_FIELDS = 26
TOTAL = BATCH * N_FIELDS
NUM_WORKERS = 32
COLS_PER_W = BATCH // NUM_WORKERS
PER_W = COLS_PER_W * N_FIELDS
LANES = 16
COL_VECS = COLS_PER_W // LANES
N_CHUNKS = 8
CHUNK = PER_W // N_CHUNKS
CVECS = CHUNK // LANES


def _sc_body(fv_hbm, idx_hbm, table_hbm, out_hbm,
             idx_v, idx1d, w_v, fv_v, out_v, gsems, osem):
    c = lax.axis_index("c")
    s = lax.axis_index("s")
    wid = s * 2 + c
    col0 = wid * COLS_PER_W
    pltpu.sync_copy(idx_hbm.at[:, pl.ds(col0, COLS_PER_W)], idx_v)

    def flat_body(t):
        j = t // COL_VECS
        ii = (t - j * COL_VECS) * LANES
        idx1d[pl.ds(t * LANES, LANES)] = idx_v[j, pl.ds(ii, LANES)]

    def fire(k):
        return pltpu.async_copy(
            table_hbm.at[idx1d.at[pl.ds(k * CHUNK, CHUNK)]],
            w_v.at[pl.ds(k * CHUNK, CHUNK)],
            gsems.at[k],
        )

    plsc.parallel_loop(0, CVECS, 1, unroll=8)(flat_body)
    gathers = [fire(0)]
    plsc.parallel_loop(CVECS, PER_W // LANES, 1, unroll=8)(flat_body)
    gathers += [fire(k) for k in range(1, N_CHUNKS)]
    pltpu.sync_copy(fv_hbm.at[:, pl.ds(col0, COLS_PER_W)], fv_v)

    for k in range(N_CHUNKS):
        gathers[k].wait()

        def mul_body(t):
            m = k * CVECS + t
            j = m // COL_VECS
            ii = (m - j * COL_VECS) * LANES
            out_v[j, pl.ds(ii, LANES)] = (
                w_v[pl.ds(m * LANES, LANES)] * fv_v[j, pl.ds(ii, LANES)])

        plsc.parallel_loop(0, CVECS, 1, unroll=8)(mul_body)

    pltpu.async_copy(out_v, out_hbm.at[:, pl.ds(col0, COLS_PER_W)],
                     osem).wait()


@jax.jit
def kernel(feature_values, feature_idx, weights_first_order):
    fvT = feature_values.T
    idxT = feature_idx.T.astype(jnp.int32)
    one = lax.optimization_barrier(jnp.float32(1.0))
    table = (weights_first_order * one).T.reshape(1000000)
    mesh = plsc.VectorSubcoreMesh(core_axis_name="c", subcore_axis_name="s")
    run = functools.partial(
        pl.kernel,
        mesh=mesh,
        out_type=jax.ShapeDtypeStruct((N_FIELDS, BATCH), jnp.float32),
        scratch_types=[
            pltpu.VMEM((N_FIELDS, COLS_PER_W), jnp.int32),
            pltpu.VMEM((PER_W,), jnp.int32),
            pltpu.VMEM((PER_W,), jnp.float32),
            pltpu.VMEM((N_FIELDS, COLS_PER_W), jnp.float32),
            pltpu.VMEM((N_FIELDS, COLS_PER_W), jnp.float32),
            pltpu.SemaphoreType.DMA((N_CHUNKS,)),
            pltpu.SemaphoreType.DMA,
        ],
    )(_sc_body)
    out = run(fvT, idxT, table)
    return out.T
